# baseline (speedup 1.0000x reference)
import jax
import jax.numpy as jnp
from jax import lax
from jax.experimental import pallas as pl
from jax.experimental.pallas import tpu as pltpu
from jax.experimental.pallas import tpu_sc as plsc

B = 8
NN = 32
D = 5
C = 32
TOT = C * C * C
NC = 2
NS = 16
NW = NC * NS
SLABS = B * NN
SLAB_PER_W = SLABS // NW
DP_FLAT = B * D * NN * 2


def _sc_weights_body(dp0_hbm, dp1_hbm, dp2_hbm, out_hbm,
                     dp0_v, dp1_v, dp2_v, cp_v, r_v, buf0, buf1,
                     sem0, sem1):
    cid = lax.axis_index("c")
    sid = lax.axis_index("s")
    wid = (sid * NC + cid).astype(jnp.int32)

    pltpu.sync_copy(dp0_hbm, dp0_v)
    pltpu.sync_copy(dp1_hbm, dp1_v)
    pltpu.sync_copy(dp2_hbm, dp2_v)

    lane = lax.broadcasted_iota(jnp.int32, (16,), 0)
    zeros16 = jnp.zeros((16,), jnp.int32)
    bufs = (buf0, buf1)
    sems = (sem0, sem1)
    dmas = [None, None]

    for k in range(SLAB_PER_W):
        slab = wid * SLAB_PER_W + k
        b = slab >> 5
        n = slab & (NN - 1)
        buf = bufs[k % 2]
        if dmas[k % 2] is not None:
            dmas[k % 2].wait()

        for v, dpv in enumerate((dp0_v, dp1_v, dp2_v)):
            for h in range(2):
                c = lane + h * 16
                acc = jnp.zeros((16,), jnp.float32)
                for dd in range(D):
                    bit = (c >> (D - 1 - dd)) & 1
                    idx = ((b * D + dd) * NN + n) * 2 + bit
                    acc = acc + plsc.load_gather(dpv, [idx])
                cp_v[pl.ds(v * C + h * 16, 16)] = acc

        def rbody(j, _):
            spl = plsc.load_gather(cp_v, [zeros16 + (C + (j >> 1))])
            r_v[pl.ds(j * 16, 16)] = spl + cp_v[pl.ds(2 * C + (j & 1) * 16, 16)]
            return 0
        lax.fori_loop(0, 64, rbody, 0, unroll=8)

        def obody(c0, _):
            spl0 = plsc.load_gather(cp_v, [zeros16 + c0])

            def inner(j, _):
                buf[pl.ds(c0 * 1024 + j * 16, 16)] = r_v[pl.ds(j * 16, 16)] + spl0
                return 0
            lax.fori_loop(0, 64, inner, 0, unroll=8)
            return 0
        lax.fori_loop(0, C, obody, 0)

        dmas[k % 2] = pltpu.async_copy(buf, out_hbm.at[slab], sems[k % 2])

    dmas[0].wait()
    dmas[1].wait()


def _ct_body(ct0_ref, ct1_ref, ct2_ref):
    t = lax.broadcasted_iota(jnp.int32, (D, TOT), 1)
    d = lax.broadcasted_iota(jnp.int32, (D, TOT), 0)
    ct0_ref[...] = (t >> (14 - d)) & 1
    ct1_ref[...] = (t >> (9 - d)) & 1
    ct2_ref[...] = (t >> (4 - d)) & 1


def kernel(dp0, dp1, dp2):
    mesh = plsc.VectorSubcoreMesh(
        core_axis_name="c", subcore_axis_name="s", num_cores=NC, num_subcores=NS)
    sc_weights = pl.kernel(
        _sc_weights_body,
        out_type=jax.ShapeDtypeStruct((SLABS, TOT), jnp.float32),
        mesh=mesh,
        compiler_params=pltpu.CompilerParams(needs_layout_passes=False),
        scratch_types=[
            pltpu.VMEM((DP_FLAT,), jnp.float32),
            pltpu.VMEM((DP_FLAT,), jnp.float32),
            pltpu.VMEM((DP_FLAT,), jnp.float32),
            pltpu.VMEM((3 * C,), jnp.float32),
            pltpu.VMEM((C * C,), jnp.float32),
            pltpu.VMEM((TOT,), jnp.float32),
            pltpu.VMEM((TOT,), jnp.float32),
            pltpu.SemaphoreType.DMA,
            pltpu.SemaphoreType.DMA,
        ],
    )
    w = sc_weights(dp0.reshape(-1), dp1.reshape(-1), dp2.reshape(-1))

    ct_shape = jax.ShapeDtypeStruct((D, TOT), jnp.int32)
    ct0, ct1, ct2 = pl.pallas_call(
        _ct_body,
        out_shape=[ct_shape, ct_shape, ct_shape],
    )()

    return ct0, ct1, ct2, w.reshape(B, NN, C, C, C)


# speedup vs baseline: 1.5107x; 1.5107x over previous
import jax
import jax.numpy as jnp
from jax import lax
from jax.experimental import pallas as pl
from jax.experimental.pallas import tpu as pltpu
from jax.experimental.pallas import tpu_sc as plsc

B = 8
NN = 32
D = 5
C = 32
TOT = C * C * C
NC = 2
NS = 16
NW = NC * NS
SLABS = B * NN
SLAB_PER_W = SLABS // NW
DP_FLAT = B * D * NN * 2


def _sc_weights_body(dp0_hbm, dp1_hbm, dp2_hbm, out_hbm,
                     dp0_v, dp1_v, dp2_v, cp_v, r_v, buf0, buf1,
                     sem0, sem1):
    cid = lax.axis_index("c")
    sid = lax.axis_index("s")
    wid = (sid * NC + cid).astype(jnp.int32)

    pltpu.sync_copy(dp0_hbm, dp0_v)
    pltpu.sync_copy(dp1_hbm, dp1_v)
    pltpu.sync_copy(dp2_hbm, dp2_v)

    lane = lax.broadcasted_iota(jnp.int32, (16,), 0)
    zeros16 = jnp.zeros((16,), jnp.int32)
    bufs = (buf0, buf1)
    sems = (sem0, sem1)
    dmas = [None, None]

    for k in range(SLAB_PER_W):
        slab = wid * SLAB_PER_W + k
        b = slab >> 5
        n = slab & (NN - 1)
        buf = bufs[k % 2]
        if dmas[k % 2] is not None:
            dmas[k % 2].wait()

        for v, dpv in enumerate((dp0_v, dp1_v, dp2_v)):
            for h in range(2):
                c = lane + h * 16
                acc = jnp.zeros((16,), jnp.float32)
                for dd in range(D):
                    bit = (c >> (D - 1 - dd)) & 1
                    idx = ((b * D + dd) * NN + n) * 2 + bit
                    acc = acc + plsc.load_gather(dpv, [idx])
                cp_v[pl.ds(v * C + h * 16, 16)] = acc

        @plsc.parallel_loop(0, 64, unroll=8)
        def _(j):
            spl = plsc.load_gather(cp_v, [zeros16 + (C + (j >> 1))])
            r_v[pl.ds(j * 16, 16)] = spl + cp_v[pl.ds(2 * C + (j & 1) * 16, 16)]

        def obody(j, _):
            rj = r_v[pl.ds(j * 16, 16)]

            @plsc.parallel_loop(0, C, unroll=8)
            def _(c0):
                spl0 = plsc.load_gather(cp_v, [zeros16 + c0])
                buf[pl.ds(c0 * 1024 + j * 16, 16)] = rj + spl0
            return 0
        lax.fori_loop(0, 64, obody, 0)

        dmas[k % 2] = pltpu.async_copy(buf, out_hbm.at[slab], sems[k % 2])

    dmas[0].wait()
    dmas[1].wait()


def _ct_body(ct0_ref, ct1_ref, ct2_ref):
    t = lax.broadcasted_iota(jnp.int32, (D, TOT), 1)
    d = lax.broadcasted_iota(jnp.int32, (D, TOT), 0)
    ct0_ref[...] = (t >> (14 - d)) & 1
    ct1_ref[...] = (t >> (9 - d)) & 1
    ct2_ref[...] = (t >> (4 - d)) & 1


def kernel(dp0, dp1, dp2):
    mesh = plsc.VectorSubcoreMesh(
        core_axis_name="c", subcore_axis_name="s", num_cores=NC, num_subcores=NS)
    sc_weights = pl.kernel(
        _sc_weights_body,
        out_type=jax.ShapeDtypeStruct((SLABS, TOT), jnp.float32),
        mesh=mesh,
        compiler_params=pltpu.CompilerParams(needs_layout_passes=False),
        scratch_types=[
            pltpu.VMEM((DP_FLAT,), jnp.float32),
            pltpu.VMEM((DP_FLAT,), jnp.float32),
            pltpu.VMEM((DP_FLAT,), jnp.float32),
            pltpu.VMEM((3 * C,), jnp.float32),
            pltpu.VMEM((C * C,), jnp.float32),
            pltpu.VMEM((TOT,), jnp.float32),
            pltpu.VMEM((TOT,), jnp.float32),
            pltpu.SemaphoreType.DMA,
            pltpu.SemaphoreType.DMA,
        ],
    )
    w = sc_weights(dp0.reshape(-1), dp1.reshape(-1), dp2.reshape(-1))

    ct_shape = jax.ShapeDtypeStruct((D, TOT), jnp.int32)
    ct0, ct1, ct2 = pl.pallas_call(
        _ct_body,
        out_shape=[ct_shape, ct_shape, ct_shape],
    )()

    return ct0, ct1, ct2, w.reshape(B, NN, C, C, C)


# speedup vs baseline: 1.6375x; 1.0839x over previous
import jax
import jax.numpy as jnp
from jax import lax
from jax.experimental import pallas as pl
from jax.experimental.pallas import tpu as pltpu
from jax.experimental.pallas import tpu_sc as plsc

B = 8
NN = 32
D = 5
C = 32
TOT = C * C * C
NC = 2
NS = 16
NW = NC * NS
SLABS = B * NN
SLAB_PER_W = SLABS // NW
DP_FLAT = B * D * NN * 2


def _sc_weights_body(dp0_hbm, dp1_hbm, dp2_hbm, out_hbm,
                     dp0_v, dp1_v, dp2_v, cp_v, r_v, buf0, buf1,
                     sem0, sem1):
    cid = lax.axis_index("c")
    sid = lax.axis_index("s")
    wid = (sid * NC + cid).astype(jnp.int32)

    pltpu.sync_copy(dp0_hbm, dp0_v)
    pltpu.sync_copy(dp1_hbm, dp1_v)
    pltpu.sync_copy(dp2_hbm, dp2_v)

    lane = lax.broadcasted_iota(jnp.int32, (16,), 0)
    zeros16 = jnp.zeros((16,), jnp.int32)
    bufs = (buf0, buf1)
    sems = (sem0, sem1)
    dmas = [None, None]

    for k in range(SLAB_PER_W):
        slab = wid * SLAB_PER_W + k
        b = slab >> 5
        n = slab & (NN - 1)
        buf = bufs[k % 2]
        if dmas[k % 2] is not None:
            dmas[k % 2].wait()

        for v, dpv in enumerate((dp0_v, dp1_v, dp2_v)):
            for h in range(2):
                c = lane + h * 16
                acc = jnp.zeros((16,), jnp.float32)
                for dd in range(D):
                    bit = (c >> (D - 1 - dd)) & 1
                    idx = ((b * D + dd) * NN + n) * 2 + bit
                    acc = acc + plsc.load_gather(dpv, [idx])
                cp_v[pl.ds(v * C + h * 16, 16)] = acc

        cp2a = cp_v[pl.ds(2 * C, 16)]
        cp2b = cp_v[pl.ds(2 * C + 16, 16)]

        @plsc.parallel_loop(0, C, unroll=4)
        def _(c1):
            spl = plsc.load_gather(cp_v, [zeros16 + (C + c1)])
            r_v[pl.ds(c1 * 32, 16)] = spl + cp2a
            r_v[pl.ds(c1 * 32 + 16, 16)] = spl + cp2b

        def obody(c0, _):
            spl0 = plsc.load_gather(cp_v, [zeros16 + c0])

            @plsc.parallel_loop(0, 64, unroll=8)
            def _(j):
                buf[pl.ds(c0 * 1024 + j * 16, 16)] = r_v[pl.ds(j * 16, 16)] + spl0
            return 0
        lax.fori_loop(0, C, obody, 0)

        dmas[k % 2] = pltpu.async_copy(buf, out_hbm.at[slab], sems[k % 2])

    dmas[0].wait()
    dmas[1].wait()


def _ct_body(ct0_ref, ct1_ref, ct2_ref):
    t = lax.broadcasted_iota(jnp.int32, (D, TOT), 1)
    d = lax.broadcasted_iota(jnp.int32, (D, TOT), 0)
    ct0_ref[...] = (t >> (14 - d)) & 1
    ct1_ref[...] = (t >> (9 - d)) & 1
    ct2_ref[...] = (t >> (4 - d)) & 1


def kernel(dp0, dp1, dp2):
    mesh = plsc.VectorSubcoreMesh(
        core_axis_name="c", subcore_axis_name="s", num_cores=NC, num_subcores=NS)
    sc_weights = pl.kernel(
        _sc_weights_body,
        out_type=jax.ShapeDtypeStruct((SLABS, TOT), jnp.float32),
        mesh=mesh,
        compiler_params=pltpu.CompilerParams(needs_layout_passes=False),
        scratch_types=[
            pltpu.VMEM((DP_FLAT,), jnp.float32),
            pltpu.VMEM((DP_FLAT,), jnp.float32),
            pltpu.VMEM((DP_FLAT,), jnp.float32),
            pltpu.VMEM((3 * C,), jnp.float32),
            pltpu.VMEM((C * C,), jnp.float32),
            pltpu.VMEM((TOT,), jnp.float32),
            pltpu.VMEM((TOT,), jnp.float32),
            pltpu.SemaphoreType.DMA,
            pltpu.SemaphoreType.DMA,
        ],
    )
    w = sc_weights(dp0.reshape(-1), dp1.reshape(-1), dp2.reshape(-1))

    ct_shape = jax.ShapeDtypeStruct((D, TOT), jnp.int32)
    ct0, ct1, ct2 = pl.pallas_call(
        _ct_body,
        out_shape=[ct_shape, ct_shape, ct_shape],
    )()

    return ct0, ct1, ct2, w.reshape(B, NN, C, C, C)


# speedup vs baseline: 2.7056x; 1.6523x over previous
import jax
import jax.numpy as jnp
from jax.experimental import pallas as pl

B = 8
NN = 32
D = 5
C = 32
TOT = C * C * C
QC = 4
CQ = C // QC


def _weights_body(dp0_ref, dp1_ref, dp2_ref, ct0_ref, ct1_ref, ct2_ref, w_ref):
    b = pl.program_id(0)
    q = pl.program_id(1)

    def cp(dp_ref, ncols):
        d = dp_ref[0]
        lo = d[:, :, 0]
        hi = d[:, :, 1]
        diff = hi - lo
        c_iota = jax.lax.broadcasted_iota(jnp.int32, (NN, ncols), 1)
        if ncols == CQ:
            c_iota = c_iota + q * CQ
        acc = jnp.zeros((NN, ncols), jnp.float32)
        for dd in range(D):
            bit = ((c_iota >> (D - 1 - dd)) & 1).astype(jnp.float32)
            acc = acc + lo[dd][:, None] + bit * diff[dd][:, None]
        return acc

    cp0 = cp(dp0_ref, CQ)
    cp1 = cp(dp1_ref, C)
    cp2 = cp(dp2_ref, C)
    s01 = cp0[:, :, None] + cp1[:, None, :]
    w_ref[0] = s01[:, :, :, None] + cp2[:, None, None, :]

    @pl.when((b == 0) & (q == 0))
    def _():
        t = jax.lax.broadcasted_iota(jnp.int32, (D, TOT), 1)
        d = jax.lax.broadcasted_iota(jnp.int32, (D, TOT), 0)
        ct0_ref[...] = (t >> (14 - d)) & 1
        ct1_ref[...] = (t >> (9 - d)) & 1
        ct2_ref[...] = (t >> (4 - d)) & 1


def kernel(dp0, dp1, dp2):
    dp_spec = pl.BlockSpec((1, D, NN, 2), lambda b, q: (b, 0, 0, 0))
    ct_spec = pl.BlockSpec((D, TOT), lambda b, q: (0, 0))
    out = pl.pallas_call(
        _weights_body,
        grid=(B, QC),
        in_specs=[dp_spec, dp_spec, dp_spec],
        out_specs=[
            ct_spec,
            ct_spec,
            ct_spec,
            pl.BlockSpec((1, NN, CQ, C, C), lambda b, q: (b, 0, q, 0, 0)),
        ],
        out_shape=[
            jax.ShapeDtypeStruct((D, TOT), jnp.int32),
            jax.ShapeDtypeStruct((D, TOT), jnp.int32),
            jax.ShapeDtypeStruct((D, TOT), jnp.int32),
            jax.ShapeDtypeStruct((B, NN, C, C, C), jnp.float32),
        ],
    )(dp0, dp1, dp2)
    return tuple(out)


# speedup vs baseline: 3.4267x; 1.2665x over previous
import jax
import jax.numpy as jnp
from jax.experimental import pallas as pl

B = 8
NN = 32
D = 5
C = 32
TOT = C * C * C


def _weights_body(dp0_ref, dp1_ref, dp2_ref, ct0_ref, ct1_ref, ct2_ref, w_ref):
    b = pl.program_id(0)

    def cp(dp_ref):
        d = dp_ref[0]
        lo = d[:, :, 0]
        hi = d[:, :, 1]
        diff = hi - lo
        c_iota = jax.lax.broadcasted_iota(jnp.int32, (NN, C), 1)
        acc = jnp.zeros((NN, C), jnp.float32)
        for dd in range(D):
            bit = ((c_iota >> (D - 1 - dd)) & 1).astype(jnp.float32)
            acc = acc + lo[dd][:, None] + bit * diff[dd][:, None]
        return acc

    cp0 = cp(dp0_ref)
    cp1 = cp(dp1_ref)
    cp2 = cp(dp2_ref)
    p12 = cp1[:, None, :, None] + cp2[:, None, None, :]
    w_ref[0] = cp0[:, :, None, None] + p12

    @pl.when(b == 0)
    def _():
        t = jax.lax.broadcasted_iota(jnp.int32, (D, TOT), 1)
        d = jax.lax.broadcasted_iota(jnp.int32, (D, TOT), 0)
        ct0_ref[...] = (t >> (14 - d)) & 1
        ct1_ref[...] = (t >> (9 - d)) & 1
        ct2_ref[...] = (t >> (4 - d)) & 1


def kernel(dp0, dp1, dp2):
    dp_spec = pl.BlockSpec((1, D, NN, 2), lambda b: (b, 0, 0, 0))
    ct_spec = pl.BlockSpec((D, TOT), lambda b: (0, 0))
    out = pl.pallas_call(
        _weights_body,
        grid=(B,),
        in_specs=[dp_spec, dp_spec, dp_spec],
        out_specs=[
            ct_spec,
            ct_spec,
            ct_spec,
            pl.BlockSpec((1, NN, C, C, C), lambda b: (b, 0, 0, 0, 0)),
        ],
        out_shape=[
            jax.ShapeDtypeStruct((D, TOT), jnp.int32),
            jax.ShapeDtypeStruct((D, TOT), jnp.int32),
            jax.ShapeDtypeStruct((D, TOT), jnp.int32),
            jax.ShapeDtypeStruct((B, NN, C, C, C), jnp.float32),
        ],
    )(dp0, dp1, dp2)
    return tuple(out)


# speedup vs baseline: 3.4983x; 1.0209x over previous
import jax
import jax.numpy as jnp
from jax.experimental import pallas as pl

B = 8
NN = 32
D = 5
C = 32
TOT = C * C * C


QC = 2
CQ = C // QC


def _weights_body(dp0_ref, dp1_ref, dp2_ref, ct0_ref, ct1_ref, ct2_ref, w_ref):
    b = pl.program_id(0)
    q = pl.program_id(1)

    def cp(dp_ref, ncols, off):
        d = dp_ref[0]
        lo = d[:, :, 0]
        hi = d[:, :, 1]
        diff = hi - lo
        c_iota = jax.lax.broadcasted_iota(jnp.int32, (NN, ncols), 1) + off
        acc = jnp.zeros((NN, ncols), jnp.float32)
        for dd in range(D):
            bit = ((c_iota >> (D - 1 - dd)) & 1).astype(jnp.float32)
            acc = acc + lo[dd][:, None] + bit * diff[dd][:, None]
        return acc

    cp0 = cp(dp0_ref, CQ, q * CQ)
    cp1 = cp(dp1_ref, C, 0)
    cp2 = cp(dp2_ref, C, 0)
    p12 = cp1[:, None, :, None] + cp2[:, None, None, :]
    w_ref[0] = cp0[:, :, None, None] + p12

    @pl.when((b == 0) & (q == 0))
    def _():
        t = jax.lax.broadcasted_iota(jnp.int32, (D, TOT), 1)
        d = jax.lax.broadcasted_iota(jnp.int32, (D, TOT), 0)
        ct0_ref[...] = (t >> (14 - d)) & 1
        ct1_ref[...] = (t >> (9 - d)) & 1
        ct2_ref[...] = (t >> (4 - d)) & 1


def kernel(dp0, dp1, dp2):
    dp_spec = pl.BlockSpec((1, D, NN, 2), lambda b, q: (b, 0, 0, 0))
    ct_spec = pl.BlockSpec((D, TOT), lambda b, q: (0, 0))
    out = pl.pallas_call(
        _weights_body,
        grid=(B, QC),
        in_specs=[dp_spec, dp_spec, dp_spec],
        out_specs=[
            ct_spec,
            ct_spec,
            ct_spec,
            pl.BlockSpec((1, NN, CQ, C, C), lambda b, q: (b, 0, q, 0, 0)),
        ],
        out_shape=[
            jax.ShapeDtypeStruct((D, TOT), jnp.int32),
            jax.ShapeDtypeStruct((D, TOT), jnp.int32),
            jax.ShapeDtypeStruct((D, TOT), jnp.int32),
            jax.ShapeDtypeStruct((B, NN, C, C, C), jnp.float32),
        ],
    )(dp0, dp1, dp2)
    return tuple(out)


# speedup vs baseline: 3.6505x; 1.0435x over previous
import jax
import jax.numpy as jnp
from jax.experimental import pallas as pl

B = 8
NN = 32
D = 5
C = 32
TOT = C * C * C


QC = 2
CQ = C // QC


def _weights_body(dps_ref, ct0_ref, ct1_ref, ct2_ref, w_ref):
    b = pl.program_id(0)
    q = pl.program_id(1)

    def cp(v, ncols, off):
        d = dps_ref[v, 0]
        lo = d[:, :, 0]
        hi = d[:, :, 1]
        diff = hi - lo
        c_iota = jax.lax.broadcasted_iota(jnp.int32, (NN, ncols), 1) + off
        acc = jnp.zeros((NN, ncols), jnp.float32)
        for dd in range(D):
            bit = ((c_iota >> (D - 1 - dd)) & 1).astype(jnp.float32)
            acc = acc + lo[dd][:, None] + bit * diff[dd][:, None]
        return acc

    cp0 = cp(0, CQ, q * CQ)
    cp1 = cp(1, C, 0)
    cp2 = cp(2, C, 0)
    p12 = cp1[:, None, :, None] + cp2[:, None, None, :]
    w_ref[0] = cp0[:, :, None, None] + p12

    @pl.when((b == 0) & (q == 0))
    def _():
        t = jax.lax.broadcasted_iota(jnp.int32, (D, TOT), 1)
        d = jax.lax.broadcasted_iota(jnp.int32, (D, TOT), 0)
        ct0_ref[...] = (t >> (14 - d)) & 1
        ct1_ref[...] = (t >> (9 - d)) & 1
        ct2_ref[...] = (t >> (4 - d)) & 1


def kernel(dp0, dp1, dp2):
    dps = jnp.stack([dp0, dp1, dp2])
    dp_spec = pl.BlockSpec((3, 1, D, NN, 2), lambda b, q: (0, b, 0, 0, 0))
    ct_spec = pl.BlockSpec((D, TOT), lambda b, q: (0, 0))
    out = pl.pallas_call(
        _weights_body,
        grid=(B, QC),
        in_specs=[dp_spec],
        out_specs=[
            ct_spec,
            ct_spec,
            ct_spec,
            pl.BlockSpec((1, NN, CQ, C, C), lambda b, q: (b, 0, q, 0, 0)),
        ],
        out_shape=[
            jax.ShapeDtypeStruct((D, TOT), jnp.int32),
            jax.ShapeDtypeStruct((D, TOT), jnp.int32),
            jax.ShapeDtypeStruct((D, TOT), jnp.int32),
            jax.ShapeDtypeStruct((B, NN, C, C, C), jnp.float32),
        ],
    )(dps)
    return tuple(out)
